# Initial kernel scaffold; baseline (speedup 1.0000x reference)
import jax
import jax.numpy as jnp
from jax.experimental import pallas as pl


def kernel(heatmap, offset, wh):
    raise NotImplementedError("write your pallas kernel here")


# speedup vs baseline: 4.8664x; 4.8664x over previous
import jax
import jax.numpy as jnp
from jax.experimental import pallas as pl
from jax.experimental.pallas import tpu as pltpu

_TOPK = 100
_SCALE = 4.0
_NEG = -1.0
_BIG = 1 << 30


def _decode_kernel(hm_ref, off_ref, wh_ref, ids_ref, sc_ref, bb_ref,
                   s_ref, rv_ref):
    C, H, W = s_ref.shape

    CHUNK = 8
    for c0 in range(0, C, CHUNK):
        hm = hm_ref[0, c0:c0 + CHUNK, :, :]
        neg_col = jnp.full((CHUNK, H, 1), _NEG, jnp.float32)
        hl = jnp.concatenate([neg_col, hm[:, :, :W - 1]], axis=2)
        hr = jnp.concatenate([hm[:, :, 1:], neg_col], axis=2)
        wm = jnp.maximum(jnp.maximum(hl, hr), hm)
        neg_row = jnp.full((CHUNK, 1, W), _NEG, jnp.float32)
        hu = jnp.concatenate([neg_row, wm[:, :H - 1, :]], axis=1)
        hd = jnp.concatenate([wm[:, 1:, :], neg_row], axis=1)
        pool = jnp.maximum(jnp.maximum(hu, hd), wm)
        s = jnp.where(pool == hm, hm, 0.0)
        s_ref[c0:c0 + CHUNK, :, :] = s
        rv_ref[c0:c0 + CHUNK, :] = jnp.max(s, axis=2)

    chio = (jax.lax.broadcasted_iota(jnp.int32, (C, H), 0) * H
            + jax.lax.broadcasted_iota(jnp.int32, (C, H), 1))
    laneH = jax.lax.broadcasted_iota(jnp.int32, (1, H), 1)
    laneW3 = jax.lax.broadcasted_iota(jnp.int32, (1, 1, W), 2)
    lane4 = jax.lax.broadcasted_iota(jnp.int32, (1, 1, 4), 2)
    ochan = jax.lax.broadcasted_iota(jnp.int32, (1, 2, 1, W), 1)
    olane = jax.lax.broadcasted_iota(jnp.int32, (1, 2, 1, W), 3)

    def body(i, _):
        val = rv_ref[:, :]
        m = jnp.max(val)
        r = jnp.min(jnp.where(val == m, chio, _BIG))
        c = r // H
        h = r % H

        srow = s_ref[pl.ds(c, 1), pl.ds(h, 1), :]
        w = jnp.min(jnp.where(srow == m, laneW3, _BIG))
        srow2 = jnp.where(laneW3 == w, _NEG, srow)
        s_ref[pl.ds(c, 1), pl.ds(h, 1), :] = srow2
        m2 = jnp.max(srow2)
        rvrow = rv_ref[pl.ds(c, 1), :]
        rv_ref[pl.ds(c, 1), :] = jnp.where(laneH == h, m2, rvrow)

        orow = off_ref[:, :, pl.ds(h, 1), :]
        wrow = wh_ref[:, :, pl.ds(h, 1), :]
        selw = olane == w
        ox = jnp.sum(jnp.where(selw & (ochan == 0), orow, 0.0))
        oy = jnp.sum(jnp.where(selw & (ochan == 1), orow, 0.0))
        bw = jnp.sum(jnp.where(selw & (ochan == 0), wrow, 0.0))
        bh = jnp.sum(jnp.where(selw & (ochan == 1), wrow, 0.0))

        xs = w.astype(jnp.float32) + ox
        ys = h.astype(jnp.float32) + oy
        hw = bw * 0.5
        hh = bh * 0.5
        x1 = (xs - hw) * _SCALE
        y1 = (ys - hh) * _SCALE
        x2 = (xs + hw) * _SCALE
        y2 = (ys + hh) * _SCALE

        sc_ref[:, pl.ds(i, 1), :] = jnp.full((1, 1, 1), m, jnp.float32)
        ids_ref[:, pl.ds(i, 1), :] = jnp.full((1, 1, 1), c.astype(jnp.float32))
        row = jnp.where(lane4 == 0, x1,
                        jnp.where(lane4 == 1, y1,
                                  jnp.where(lane4 == 2, x2, y2)))
        bb_ref[:, pl.ds(i, 1), :] = row
        return 0

    jax.lax.fori_loop(0, _TOPK, body, 0)


def _build_call(B, C, H, W, interpret=False):
    return pl.pallas_call(
        _decode_kernel,
        grid=(B,),
        in_specs=[
            pl.BlockSpec((1, C, H, W), lambda b: (b, 0, 0, 0)),
            pl.BlockSpec((1, 2, H, W), lambda b: (b, 0, 0, 0)),
            pl.BlockSpec((1, 2, H, W), lambda b: (b, 0, 0, 0)),
        ],
        out_specs=[
            pl.BlockSpec((1, _TOPK, 1), lambda b: (b, 0, 0)),
            pl.BlockSpec((1, _TOPK, 1), lambda b: (b, 0, 0)),
            pl.BlockSpec((1, _TOPK, 4), lambda b: (b, 0, 0)),
        ],
        out_shape=[
            jax.ShapeDtypeStruct((B, _TOPK, 1), jnp.float32),
            jax.ShapeDtypeStruct((B, _TOPK, 1), jnp.float32),
            jax.ShapeDtypeStruct((B, _TOPK, 4), jnp.float32),
        ],
        scratch_shapes=[
            pltpu.VMEM((C, H, W), jnp.float32),
            pltpu.VMEM((C, H), jnp.float32),
        ],
        compiler_params=pltpu.CompilerParams(
            dimension_semantics=("parallel",)),
        interpret=interpret,
    )


@jax.jit
def kernel(heatmap, offset, wh):
    B, C, H, W = heatmap.shape
    ids, scores, bboxes = _build_call(B, C, H, W)(heatmap, offset, wh)
    return ids, scores, bboxes


# speedup vs baseline: 4.8870x; 1.0042x over previous
import jax
import jax.numpy as jnp
from jax.experimental import pallas as pl
from jax.experimental.pallas import tpu as pltpu

_TOPK = 100
_SCALE = 4.0
_NEG = -1.0
_BIG = 1 << 30


def _decode_kernel(hm_ref, off_ref, wh_ref, ids_ref, sc_ref, bb_ref, s_ref):
    C, H, W = s_ref.shape

    CHUNK = 8
    row_maxes = []
    for c0 in range(0, C, CHUNK):
        hm = hm_ref[0, c0:c0 + CHUNK, :, :]
        neg_col = jnp.full((CHUNK, H, 1), _NEG, jnp.float32)
        hl = jnp.concatenate([neg_col, hm[:, :, :W - 1]], axis=2)
        hr = jnp.concatenate([hm[:, :, 1:], neg_col], axis=2)
        wm = jnp.maximum(jnp.maximum(hl, hr), hm)
        neg_row = jnp.full((CHUNK, 1, W), _NEG, jnp.float32)
        hu = jnp.concatenate([neg_row, wm[:, :H - 1, :]], axis=1)
        hd = jnp.concatenate([wm[:, 1:, :], neg_row], axis=1)
        pool = jnp.maximum(jnp.maximum(hu, hd), wm)
        s = jnp.where(pool == hm, hm, 0.0)
        s_ref[c0:c0 + CHUNK, :, :] = s
        row_maxes.append(jnp.max(s, axis=2))
    rv0 = jnp.concatenate(row_maxes, axis=0)

    chio = (jax.lax.broadcasted_iota(jnp.int32, (C, H), 0) * H
            + jax.lax.broadcasted_iota(jnp.int32, (C, H), 1))
    laneW3 = jax.lax.broadcasted_iota(jnp.int32, (1, 1, W), 2)
    lane4 = jax.lax.broadcasted_iota(jnp.int32, (1, 1, 4), 2)
    ochan = jax.lax.broadcasted_iota(jnp.int32, (1, 2, 1, W), 1)
    olane = jax.lax.broadcasted_iota(jnp.int32, (1, 2, 1, W), 3)

    def body(i, rv):
        m = jnp.max(rv)
        r = jnp.min(jnp.where(rv == m, chio, _BIG))
        c = r // H
        h = r % H

        srow = s_ref[pl.ds(c, 1), pl.ds(h, 1), :]
        w = jnp.min(jnp.where(srow == m, laneW3, _BIG))
        srow2 = jnp.where(laneW3 == w, _NEG, srow)
        s_ref[pl.ds(c, 1), pl.ds(h, 1), :] = srow2
        m2 = jnp.max(srow2)
        rv2 = jnp.where(chio == r, m2, rv)

        orow = off_ref[:, :, pl.ds(h, 1), :]
        wrow = wh_ref[:, :, pl.ds(h, 1), :]
        selw = olane == w
        ox = jnp.sum(jnp.where(selw & (ochan == 0), orow, 0.0))
        oy = jnp.sum(jnp.where(selw & (ochan == 1), orow, 0.0))
        bw = jnp.sum(jnp.where(selw & (ochan == 0), wrow, 0.0))
        bh = jnp.sum(jnp.where(selw & (ochan == 1), wrow, 0.0))

        xs = w.astype(jnp.float32) + ox
        ys = h.astype(jnp.float32) + oy
        hw = bw * 0.5
        hh = bh * 0.5
        x1 = (xs - hw) * _SCALE
        y1 = (ys - hh) * _SCALE
        x2 = (xs + hw) * _SCALE
        y2 = (ys + hh) * _SCALE

        sc_ref[:, pl.ds(i, 1), :] = jnp.full((1, 1, 1), m, jnp.float32)
        ids_ref[:, pl.ds(i, 1), :] = jnp.full((1, 1, 1), c.astype(jnp.float32))
        row = jnp.where(lane4 == 0, x1,
                        jnp.where(lane4 == 1, y1,
                                  jnp.where(lane4 == 2, x2, y2)))
        bb_ref[:, pl.ds(i, 1), :] = row
        return rv2

    jax.lax.fori_loop(0, _TOPK, body, rv0)


def _build_call(B, C, H, W, interpret=False):
    return pl.pallas_call(
        _decode_kernel,
        grid=(B,),
        in_specs=[
            pl.BlockSpec((1, C, H, W), lambda b: (b, 0, 0, 0)),
            pl.BlockSpec((1, 2, H, W), lambda b: (b, 0, 0, 0)),
            pl.BlockSpec((1, 2, H, W), lambda b: (b, 0, 0, 0)),
        ],
        out_specs=[
            pl.BlockSpec((1, _TOPK, 1), lambda b: (b, 0, 0)),
            pl.BlockSpec((1, _TOPK, 1), lambda b: (b, 0, 0)),
            pl.BlockSpec((1, _TOPK, 4), lambda b: (b, 0, 0)),
        ],
        out_shape=[
            jax.ShapeDtypeStruct((B, _TOPK, 1), jnp.float32),
            jax.ShapeDtypeStruct((B, _TOPK, 1), jnp.float32),
            jax.ShapeDtypeStruct((B, _TOPK, 4), jnp.float32),
        ],
        scratch_shapes=[
            pltpu.VMEM((C, H, W), jnp.float32),
        ],
        compiler_params=pltpu.CompilerParams(
            dimension_semantics=("parallel",)),
        interpret=interpret,
    )


@jax.jit
def kernel(heatmap, offset, wh):
    B, C, H, W = heatmap.shape
    ids, scores, bboxes = _build_call(B, C, H, W)(heatmap, offset, wh)
    return ids, scores, bboxes


# speedup vs baseline: 4.9193x; 1.0066x over previous
import jax
import jax.numpy as jnp
from jax.experimental import pallas as pl
from jax.experimental.pallas import tpu as pltpu

_TOPK = 100
_SCALE = 4.0
_NEG = -1.0
_BIG = 1 << 30


def _decode_kernel(hm_ref, off_ref, wh_ref, ids_ref, sc_ref, bb_ref,
                   s_ref, mcol_ref, rcol_ref, wcol_ref):
    C, H, W = s_ref.shape

    CHUNK = 8
    row_maxes = []
    for c0 in range(0, C, CHUNK):
        hm = hm_ref[0, c0:c0 + CHUNK, :, :]
        neg_col = jnp.full((CHUNK, H, 1), _NEG, jnp.float32)
        hl = jnp.concatenate([neg_col, hm[:, :, :W - 1]], axis=2)
        hr = jnp.concatenate([hm[:, :, 1:], neg_col], axis=2)
        wm = jnp.maximum(jnp.maximum(hl, hr), hm)
        neg_row = jnp.full((CHUNK, 1, W), _NEG, jnp.float32)
        hu = jnp.concatenate([neg_row, wm[:, :H - 1, :]], axis=1)
        hd = jnp.concatenate([wm[:, 1:, :], neg_row], axis=1)
        pool = jnp.maximum(jnp.maximum(hu, hd), wm)
        s = jnp.where(pool == hm, hm, 0.0)
        s_ref[c0:c0 + CHUNK, :, :] = s
        row_maxes.append(jnp.max(s, axis=2))
    rv0 = jnp.concatenate(row_maxes, axis=0)

    chio = (jax.lax.broadcasted_iota(jnp.int32, (C, H), 0) * H
            + jax.lax.broadcasted_iota(jnp.int32, (C, H), 1))
    laneW3 = jax.lax.broadcasted_iota(jnp.int32, (1, 1, W), 2)

    def body(i, rv):
        m = jnp.max(rv)
        r = jnp.min(jnp.where(rv == m, chio, _BIG))
        c = r // H
        h = r % H
        srow = s_ref[pl.ds(c, 1), pl.ds(h, 1), :]
        w = jnp.min(jnp.where(srow == m, laneW3, _BIG))
        srow2 = jnp.where(laneW3 == w, _NEG, srow)
        s_ref[pl.ds(c, 1), pl.ds(h, 1), :] = srow2
        m2 = jnp.max(srow2)
        mcol_ref[pl.ds(i, 1), :] = jnp.full((1, 1), m, jnp.float32)
        rcol_ref[pl.ds(i, 1), :] = jnp.full((1, 1), r, jnp.int32)
        wcol_ref[pl.ds(i, 1), :] = jnp.full((1, 1), w, jnp.int32)
        return jnp.where(chio == r, m2, rv)

    jax.lax.fori_loop(0, _TOPK, body, rv0)

    mcol = mcol_ref[:, :]
    rcol = rcol_ref[:, :]
    wcol = wcol_ref[:, :]
    ccol = rcol // H
    hcol = rcol % H

    colj = jax.lax.broadcasted_iota(jnp.int32, (_TOPK, W), 1)
    ohh = jnp.where(hcol == colj, 1.0, 0.0)
    ohw = jnp.where(wcol == colj, 1.0, 0.0)

    def gather_plane(plane):
        rows = jnp.dot(ohh, plane, preferred_element_type=jnp.float32)
        return jnp.sum(rows * ohw, axis=1, keepdims=True)

    ox = gather_plane(off_ref[0, 0, :, :])
    oy = gather_plane(off_ref[0, 1, :, :])
    bw = gather_plane(wh_ref[0, 0, :, :])
    bh = gather_plane(wh_ref[0, 1, :, :])

    xs = wcol.astype(jnp.float32) + ox
    ys = hcol.astype(jnp.float32) + oy
    hw = bw * 0.5
    hh = bh * 0.5
    x1 = (xs - hw) * _SCALE
    y1 = (ys - hh) * _SCALE
    x2 = (xs + hw) * _SCALE
    y2 = (ys + hh) * _SCALE

    sc_ref[0, :, :] = mcol
    ids_ref[0, :, :] = ccol.astype(jnp.float32)
    bb_ref[0, :, :] = jnp.concatenate([x1, y1, x2, y2], axis=1)


def _build_call(B, C, H, W, interpret=False):
    return pl.pallas_call(
        _decode_kernel,
        grid=(B,),
        in_specs=[
            pl.BlockSpec((1, C, H, W), lambda b: (b, 0, 0, 0)),
            pl.BlockSpec((1, 2, H, W), lambda b: (b, 0, 0, 0)),
            pl.BlockSpec((1, 2, H, W), lambda b: (b, 0, 0, 0)),
        ],
        out_specs=[
            pl.BlockSpec((1, _TOPK, 1), lambda b: (b, 0, 0)),
            pl.BlockSpec((1, _TOPK, 1), lambda b: (b, 0, 0)),
            pl.BlockSpec((1, _TOPK, 4), lambda b: (b, 0, 0)),
        ],
        out_shape=[
            jax.ShapeDtypeStruct((B, _TOPK, 1), jnp.float32),
            jax.ShapeDtypeStruct((B, _TOPK, 1), jnp.float32),
            jax.ShapeDtypeStruct((B, _TOPK, 4), jnp.float32),
        ],
        scratch_shapes=[
            pltpu.VMEM((C, H, W), jnp.float32),
            pltpu.VMEM((_TOPK, 1), jnp.float32),
            pltpu.VMEM((_TOPK, 1), jnp.int32),
            pltpu.VMEM((_TOPK, 1), jnp.int32),
        ],
        compiler_params=pltpu.CompilerParams(
            dimension_semantics=("parallel",)),
        interpret=interpret,
    )


@jax.jit
def kernel(heatmap, offset, wh):
    B, C, H, W = heatmap.shape
    ids, scores, bboxes = _build_call(B, C, H, W)(heatmap, offset, wh)
    return ids, scores, bboxes
